# baseline (device time: 85232 ns/iter reference)
import jax
import jax.numpy as jnp
from jax import lax
from jax.experimental import pallas as pl
from jax.experimental.pallas import tpu as pltpu

N_DEV = 32
N_TOK = 512
D_IN = 256
D_OUT = 512
N_EXP = 128
E_LOCAL = N_EXP // N_DEV
ROWS_PER = N_TOK // N_DEV


def kernel(x, router_W, route_idx, expert_W, shared_W):
    def body(x_ref, rw_ref, idx_ref, ew_ref, sw_ref, out_ref,
             acc_ref, recv_ref, send_sems, recv_sems):
        i = lax.axis_index("i")
        left = lax.rem(i + N_DEV - 1, N_DEV)
        right = lax.rem(i + 1, N_DEV)

        barrier = pltpu.get_barrier_semaphore()
        for nbr in (left, right):
            pl.semaphore_signal(barrier, inc=1, device_id=(nbr,),
                                device_id_type=pl.DeviceIdType.MESH)
        pl.semaphore_wait(barrier, 2)

        xv = x_ref[...]
        route = idx_ref[...]

        scores = jnp.dot(xv, rw_ref[...], preferred_element_type=jnp.float32)
        m = jnp.max(scores, axis=-1, keepdims=True)
        e = jnp.exp(scores - m)
        denom = jnp.sum(e, axis=-1, keepdims=True)
        onehot = route == lax.broadcasted_iota(jnp.int32, (N_TOK, N_EXP), 1)
        sel = jnp.sum(jnp.where(onehot, e, 0.0), axis=-1, keepdims=True)
        gate = sel / denom

        acc = jnp.zeros((N_TOK, D_OUT), jnp.float32)
        for el in range(E_LOCAL):
            ge = i * E_LOCAL + el
            mask = (route == ge).astype(jnp.float32)
            acc = acc + jnp.dot(xv * mask, ew_ref[el],
                                preferred_element_type=jnp.float32)
        acc_ref[...] = acc * gate

        for s in range(N_DEV - 1):
            send_chunk = lax.rem(i + 2 * N_DEV - 1 - s, N_DEV)
            rdma = pltpu.make_async_remote_copy(
                src_ref=acc_ref.at[pl.ds(send_chunk * ROWS_PER, ROWS_PER), :],
                dst_ref=recv_ref.at[s],
                send_sem=send_sems.at[s],
                recv_sem=recv_sems.at[s],
                device_id=(right,),
                device_id_type=pl.DeviceIdType.MESH,
            )
            rdma.start()
            rdma.wait()
            recv_chunk = lax.rem(i + 2 * N_DEV - 2 - s, N_DEV)
            rows = pl.ds(recv_chunk * ROWS_PER, ROWS_PER)
            acc_ref[rows, :] = acc_ref[rows, :] + recv_ref[s]

        my_rows = pl.ds(i * ROWS_PER, ROWS_PER)
        shared = jnp.dot(x_ref[my_rows, :], sw_ref[...],
                         preferred_element_type=jnp.float32)
        out_ref[...] = shared + acc_ref[my_rows, :]

        def _exit(sem2):
            for nbr in (left, right):
                pl.semaphore_signal(sem2, inc=1, device_id=(nbr,),
                                    device_id_type=pl.DeviceIdType.MESH)
            pl.semaphore_wait(sem2, 2)

        pl.run_scoped(_exit, sem2=pltpu.SemaphoreType.REGULAR)

    return pl.pallas_call(
        body,
        out_shape=jax.ShapeDtypeStruct((ROWS_PER, D_OUT), jnp.float32),
        in_specs=[pl.BlockSpec(memory_space=pltpu.VMEM)] * 5,
        out_specs=pl.BlockSpec(memory_space=pltpu.VMEM),
        scratch_shapes=[
            pltpu.VMEM((N_TOK, D_OUT), jnp.float32),
            pltpu.VMEM((N_DEV - 1, ROWS_PER, D_OUT), jnp.float32),
            pltpu.SemaphoreType.DMA((N_DEV - 1,)),
            pltpu.SemaphoreType.DMA((N_DEV - 1,)),
        ],
        compiler_params=pltpu.CompilerParams(collective_id=0),
    )(x, router_W, route_idx, expert_W, shared_W)
